# Initial kernel scaffold; baseline (speedup 1.0000x reference)
import jax
import jax.numpy as jnp
from jax.experimental import pallas as pl


def kernel(features, adj, W1, b1, W2, b2):
    raise NotImplementedError("write your pallas kernel here")


# speedup vs baseline: 13.2218x; 13.2218x over previous
import functools

import jax
import jax.numpy as jnp
from jax.experimental import pallas as pl

K1 = 33
EPS = 0.3
FUSION = 0.1

_HIGH = jax.lax.Precision.DEFAULT
_INT_MIN = -2147483648


def _linear_kernel(x_ref, w_ref, b_ref, o_ref):
    o_ref[...] = (
        jnp.dot(x_ref[...], w_ref[...], precision=_HIGH,
                preferred_element_type=jnp.float32)
        + b_ref[...]
    )


def _linear(x, w, b):
    n, d = x.shape
    return pl.pallas_call(
        _linear_kernel,
        out_shape=jax.ShapeDtypeStruct((n, d), jnp.float32),
    )(x, w, b.reshape(1, d))


def _adj_gemm_kernel(adj_ref, y_ref, o_ref, *, relu):
    acc = jax.lax.dot_general(
        adj_ref[...], y_ref[...], (((1,), (0,)), ((), ())),
        precision=_HIGH, preferred_element_type=jnp.float32)
    o_ref[...] = jnp.maximum(acc, 0.0) if relu else acc


def _adj_gemm(adj, y, relu, blk):
    n, d = y.shape
    return pl.pallas_call(
        functools.partial(_adj_gemm_kernel, relu=relu),
        grid=(n // blk,),
        in_specs=[
            pl.BlockSpec((blk, n), lambda i: (i, 0)),
            pl.BlockSpec((n, d), lambda i: (0, 0)),
        ],
        out_specs=pl.BlockSpec((blk, d), lambda i: (i, 0)),
        out_shape=jax.ShapeDtypeStruct((n, d), jnp.float32),
    )(adj, y)


def _bits_to_f32(u):
    bits = jnp.where(u < 0, u ^ jnp.int32(_INT_MIN), ~u)
    return jax.lax.bitcast_convert_type(bits, jnp.float32)


def _select_kernel(e_blk_ref, et_ref, adj_ref, o_ref):
    sim = jax.lax.dot_general(
        e_blk_ref[...], et_ref[...], (((1,), (0,)), ((), ())),
        precision=_HIGH, preferred_element_type=jnp.float32)

    blk = sim.shape[0]

    def body(i, t):
        bit = jnp.left_shift(jnp.int32(1), jnp.int32(31) - i)
        cand = t | bit
        cand_f = _bits_to_f32(cand)
        cnt = jnp.sum((sim >= cand_f).astype(jnp.float32), axis=1,
                      keepdims=True)
        return jnp.where(cnt >= float(K1), cand, t)

    t0 = jnp.zeros((blk, 1), jnp.int32)
    t = jax.lax.fori_loop(0, 32, body, t0)
    thresh = _bits_to_f32(t)

    keep = (sim >= thresh) & (sim > EPS)
    o_ref[...] = jnp.where(keep, FUSION * sim, 0.0) + (1.0 - FUSION) * adj_ref[...]


def _select(e, e_t, adj, blk):
    n, d = e.shape
    return pl.pallas_call(
        _select_kernel,
        grid=(n // blk,),
        in_specs=[
            pl.BlockSpec((blk, d), lambda i: (i, 0)),
            pl.BlockSpec((d, n), lambda i: (0, 0)),
            pl.BlockSpec((blk, n), lambda i: (i, 0)),
        ],
        out_specs=pl.BlockSpec((blk, n), lambda i: (i, 0)),
        out_shape=jax.ShapeDtypeStruct((n, n), jnp.float32),
    )(e, e_t, adj)


def kernel(features, adj, W1, b1, W2, b2):
    n, d = features.shape
    blk = min(128, n)
    y1 = _linear(features, W1, b1)
    h = _adj_gemm(adj, y1, relu=True, blk=blk)
    y2 = _linear(h, W2, b2)
    e = _adj_gemm(adj, y2, relu=False, blk=blk)
    final_adj = _select(e, e.T, adj, blk=blk)
    return e, final_adj
